# Initial kernel scaffold; baseline (speedup 1.0000x reference)
import jax
import jax.numpy as jnp
from jax.experimental import pallas as pl


def kernel(epes_stat_flow, epes_dyn_flow, dynamicness_scores, moving_average_importance, bias_counter):
    raise NotImplementedError("write your pallas kernel here")


# speedup vs baseline: 27.2088x; 27.2088x over previous
import functools

import jax
import jax.numpy as jnp
from jax import lax
from jax.experimental import pallas as pl
from jax.experimental.pallas import tpu as pltpu
from jax.experimental.pallas import tpu_sc as plsc

RES = 100000
N = 8388608
V0 = 0.0
VR = 1.0
START_VALUE = 0.5
UPDATE_WEIGHT = 1.0 / 37500000.0
CW = float((1.0 - UPDATE_WEIGHT) ** float(N))

NC = 2
NS = 16
NW = NC * NS
NPW = N // NW
CH = 8192
NCHUNK = NPW // CH
CROWS = CH // 128
HP = 100096
ZS = HP // NS

P2R = 800
P2C = 128


def _sc_hist_body(stat_hbm, dyn_hbm, score_hbm, out_hbm, hist_sh,
                  stat_v, dyn_v, score_v, vals_v, idx_v, zbuf):
    c = lax.axis_index("c")
    s = lax.axis_index("s")
    wid = s * NC + c

    @pl.loop(0, ZS // 16)
    def _zero(i):
        zbuf[pl.ds(i * 16, 16)] = jnp.zeros((16,), jnp.float32)

    pltpu.sync_copy(zbuf, hist_sh.at[pl.ds(s * ZS, ZS)])
    plsc.subcore_barrier()

    base = wid * NPW

    @pl.loop(0, NCHUNK)
    def _chunk(k):
        off = base + k * CH
        pltpu.sync_copy(stat_hbm.at[pl.ds(off, CH)], stat_v)
        pltpu.sync_copy(dyn_hbm.at[pl.ds(off, CH)], dyn_v)
        pltpu.sync_copy(score_hbm.at[pl.ds(off, CH)], score_v)

        @pl.loop(0, CH // 16, unroll=8)
        def _vec(i):
            st = stat_v[pl.ds(i * 16, 16)]
            dy = dyn_v[pl.ds(i * 16, 16)]
            sc = score_v[pl.ds(i * 16, 16)]
            vals = st - dy
            idxi = (sc * float(RES)).astype(jnp.int32)
            idxi = jnp.minimum(idxi, RES - 1)
            vals_v[pl.ds(i * 16, 16)] = vals
            idx_v[pl.ds(i * 16, 16)] = idxi

        pltpu.sync_copy(vals_v, hist_sh.at[idx_v], add=True)

    plsc.subcore_barrier()

    pltpu.sync_copy(hist_sh.at[pl.ds(s * ZS, ZS)], zbuf)
    pltpu.sync_copy(zbuf, out_hbm.at[pl.ds(c * HP + s * ZS, ZS)])


@functools.cache
def _sc_hist():
    return pl.kernel(
        _sc_hist_body,
        out_type=jax.ShapeDtypeStruct((NC * HP,), jnp.float32),
        mesh=plsc.VectorSubcoreMesh(core_axis_name="c", subcore_axis_name="s",
                                    num_cores=NC, num_subcores=NS),
        scratch_types=[
            pltpu.VMEM_SHARED((HP,), jnp.float32),
            pltpu.VMEM((CH,), jnp.float32),
            pltpu.VMEM((CH,), jnp.float32),
            pltpu.VMEM((CH,), jnp.float32),
            pltpu.VMEM((CH,), jnp.float32),
            pltpu.VMEM((CH,), jnp.int32),
            pltpu.VMEM((ZS,), jnp.float32),
        ],
    )


def _shift_right_cols(x, s):
    return jnp.concatenate(
        [jnp.zeros((x.shape[0], s), jnp.float32), x[:, : x.shape[1] - s]], axis=1)


def _shift_down_rows(x, s):
    return jnp.concatenate(
        [jnp.zeros((s, x.shape[1]), jnp.float32), x[: x.shape[0] - s, :]], axis=0)


def _tc_thresh_body(parts_ref, mov_ref, bias_ref, out_ref):
    h = parts_ref[0] + parts_ref[1]
    imp = mov_ref[...] * CW + (1.0 - CW) * h

    x = imp
    for sft in (1, 2, 4, 8, 16, 32, 64):
        x = x + _shift_right_cols(x, sft)
    rowtot = x[:, P2C - 1:P2C]

    t = _shift_down_rows(rowtot, 1)
    for sft in (1, 2, 4, 8, 16, 32, 64, 128, 256, 512):
        t = t + _shift_down_rows(t, sft)

    prefix = x + t

    flat = (lax.broadcasted_iota(jnp.int32, (P2R, P2C), 0) * P2C
            + lax.broadcasted_iota(jnp.int32, (P2R, P2C), 1))
    valid = flat < RES
    pv = jnp.where(valid, prefix, jnp.inf)
    best = jnp.minimum(jnp.min(pv), 0.0)

    maskv = valid & (prefix == best)
    cnt = (jnp.sum(maskv.astype(jnp.float32))
           + jnp.where(best == 0.0, 1.0, 0.0))
    sidx = jnp.sum(jnp.where(maskv, (flat + 1).astype(jnp.float32), 0.0))
    avg = sidx / cnt
    thr = avg * VR / float(RES) + V0

    new_bias = bias_ref[0, 0] * CW + (1.0 - CW)
    out_ref[0, 0] = jnp.where(new_bias > 0.0, thr, jnp.float32(START_VALUE))


def kernel(epes_stat_flow, epes_dyn_flow, dynamicness_scores,
           moving_average_importance, bias_counter):
    parts = _sc_hist()(epes_stat_flow, epes_dyn_flow, dynamicness_scores)
    parts = parts.reshape(NC, HP)

    parts_p = jnp.pad(parts[:, :RES], ((0, 0), (0, P2R * P2C - RES)))
    parts_p = parts_p.reshape(NC, P2R, P2C)
    mov_p = jnp.pad(moving_average_importance, (0, P2R * P2C - RES))
    mov_p = mov_p.reshape(P2R, P2C)
    bias = jnp.reshape(bias_counter, (1, 1))

    out = pl.pallas_call(
        _tc_thresh_body,
        out_shape=jax.ShapeDtypeStruct((1, 1), jnp.float32),
        in_specs=[
            pl.BlockSpec(memory_space=pltpu.VMEM),
            pl.BlockSpec(memory_space=pltpu.VMEM),
            pl.BlockSpec(memory_space=pltpu.SMEM),
        ],
        out_specs=pl.BlockSpec(memory_space=pltpu.SMEM),
    )(parts_p, mov_p, bias)
    return out[0, 0]


# speedup vs baseline: 53.5723x; 1.9689x over previous
import functools

import jax
import jax.numpy as jnp
from jax import lax
from jax.experimental import pallas as pl
from jax.experimental.pallas import tpu as pltpu
from jax.experimental.pallas import tpu_sc as plsc

RES = 100000
N = 8388608
V0 = 0.0
VR = 1.0
START_VALUE = 0.5
UPDATE_WEIGHT = 1.0 / 37500000.0
CW = float((1.0 - UPDATE_WEIGHT) ** float(N))

NC = 2
NS = 16
NW = NC * NS
NPW = N // NW
CH = 8192
NCHUNK = NPW // CH
CROWS = CH // 128
HP = 100096
ZS = HP // NS

P2R = 800
P2C = 128


def _sc_hist_body(stat_hbm, dyn_hbm, score_hbm, out_hbm, hist_sh,
                  stat_v, dyn_v, score_v, vals_v, idx_v, zbuf,
                  in_sems, sc_sems):
    c = lax.axis_index("c")
    s = lax.axis_index("s")
    wid = s * NC + c

    @pl.loop(0, ZS // 16)
    def _zero(i):
        zbuf[pl.ds(i * 16, 16)] = jnp.zeros((16,), jnp.float32)

    pltpu.sync_copy(zbuf, hist_sh.at[pl.ds(s * ZS, ZS)])
    plsc.subcore_barrier()

    base = wid * NPW
    inputs = (stat_hbm, dyn_hbm, score_hbm)

    def start_inputs(b, k):
        off = base + k * CH
        for src, dst in zip(inputs, (stat_v[b], dyn_v[b], score_v[b])):
            pltpu.async_copy(src.at[pl.ds(off, CH)], dst, in_sems[b])

    def wait_inputs(b, k):
        off = base + k * CH
        for src, dst in zip(inputs, (stat_v[b], dyn_v[b], score_v[b])):
            pltpu.make_async_copy(src.at[pl.ds(off, CH)], dst, in_sems[b]).wait()

    def wait_scatter(b):
        pltpu.make_async_copy(
            vals_v[b], hist_sh.at[idx_v[b]], sc_sems[b]).wait()

    start_inputs(0, 0)
    start_inputs(1, 1)

    @pl.loop(0, NCHUNK // 2)
    def _round(o):
        for b in range(2):
            k = o * 2 + b
            wait_inputs(b, k)

            @pl.when(k >= 2)
            def _():
                wait_scatter(b)

            @pl.loop(0, CH // 16, unroll=8)
            def _vec(i):
                st = stat_v[b][pl.ds(i * 16, 16)]
                dy = dyn_v[b][pl.ds(i * 16, 16)]
                sc = score_v[b][pl.ds(i * 16, 16)]
                vals = st - dy
                idxi = (sc * float(RES)).astype(jnp.int32)
                idxi = jnp.minimum(idxi, RES - 1)
                vals_v[b][pl.ds(i * 16, 16)] = vals
                idx_v[b][pl.ds(i * 16, 16)] = idxi

            pltpu.async_copy(vals_v[b], hist_sh.at[idx_v[b]], sc_sems[b],
                             add=True)

            @pl.when(k + 2 < NCHUNK)
            def _():
                start_inputs(b, k + 2)

    wait_scatter(0)
    wait_scatter(1)
    plsc.subcore_barrier()

    pltpu.sync_copy(hist_sh.at[pl.ds(s * ZS, ZS)], zbuf)
    pltpu.sync_copy(zbuf, out_hbm.at[pl.ds(c * HP + s * ZS, ZS)])


@functools.cache
def _sc_hist():
    return pl.kernel(
        _sc_hist_body,
        out_type=jax.ShapeDtypeStruct((NC * HP,), jnp.float32),
        mesh=plsc.VectorSubcoreMesh(core_axis_name="c", subcore_axis_name="s",
                                    num_cores=NC, num_subcores=NS),
        scratch_types=[
            pltpu.VMEM_SHARED((HP,), jnp.float32),
            [pltpu.VMEM((CH,), jnp.float32) for _ in range(2)],
            [pltpu.VMEM((CH,), jnp.float32) for _ in range(2)],
            [pltpu.VMEM((CH,), jnp.float32) for _ in range(2)],
            [pltpu.VMEM((CH,), jnp.float32) for _ in range(2)],
            [pltpu.VMEM((CH,), jnp.int32) for _ in range(2)],
            pltpu.VMEM((ZS,), jnp.float32),
            [pltpu.SemaphoreType.DMA for _ in range(2)],
            [pltpu.SemaphoreType.DMA for _ in range(2)],
        ],
    )


def _shift_right_cols(x, s):
    return jnp.concatenate(
        [jnp.zeros((x.shape[0], s), jnp.float32), x[:, : x.shape[1] - s]], axis=1)


def _shift_down_rows(x, s):
    return jnp.concatenate(
        [jnp.zeros((s, x.shape[1]), jnp.float32), x[: x.shape[0] - s, :]], axis=0)


def _tc_thresh_body(parts_ref, mov_ref, bias_ref, out_ref):
    h = parts_ref[0] + parts_ref[1]
    imp = mov_ref[...] * CW + (1.0 - CW) * h

    x = imp
    for sft in (1, 2, 4, 8, 16, 32, 64):
        x = x + _shift_right_cols(x, sft)
    rowtot = x[:, P2C - 1:P2C]

    t = _shift_down_rows(rowtot, 1)
    for sft in (1, 2, 4, 8, 16, 32, 64, 128, 256, 512):
        t = t + _shift_down_rows(t, sft)

    prefix = x + t

    flat = (lax.broadcasted_iota(jnp.int32, (P2R, P2C), 0) * P2C
            + lax.broadcasted_iota(jnp.int32, (P2R, P2C), 1))
    valid = flat < RES
    pv = jnp.where(valid, prefix, jnp.inf)
    best = jnp.minimum(jnp.min(pv), 0.0)

    maskv = valid & (prefix == best)
    cnt = (jnp.sum(maskv.astype(jnp.float32))
           + jnp.where(best == 0.0, 1.0, 0.0))
    sidx = jnp.sum(jnp.where(maskv, (flat + 1).astype(jnp.float32), 0.0))
    avg = sidx / cnt
    thr = avg * VR / float(RES) + V0

    new_bias = bias_ref[0, 0] * CW + (1.0 - CW)
    out_ref[0, 0] = jnp.where(new_bias > 0.0, thr, jnp.float32(START_VALUE))


def kernel(epes_stat_flow, epes_dyn_flow, dynamicness_scores,
           moving_average_importance, bias_counter):
    parts = _sc_hist()(epes_stat_flow, epes_dyn_flow, dynamicness_scores)
    parts = parts.reshape(NC, HP)

    parts_p = jnp.pad(parts[:, :RES], ((0, 0), (0, P2R * P2C - RES)))
    parts_p = parts_p.reshape(NC, P2R, P2C)
    mov_p = jnp.pad(moving_average_importance, (0, P2R * P2C - RES))
    mov_p = mov_p.reshape(P2R, P2C)
    bias = jnp.reshape(bias_counter, (1, 1))

    out = pl.pallas_call(
        _tc_thresh_body,
        out_shape=jax.ShapeDtypeStruct((1, 1), jnp.float32),
        in_specs=[
            pl.BlockSpec(memory_space=pltpu.VMEM),
            pl.BlockSpec(memory_space=pltpu.VMEM),
            pl.BlockSpec(memory_space=pltpu.SMEM),
        ],
        out_specs=pl.BlockSpec(memory_space=pltpu.SMEM),
    )(parts_p, mov_p, bias)
    return out[0, 0]


# speedup vs baseline: 53.8547x; 1.0053x over previous
import functools

import jax
import jax.numpy as jnp
from jax import lax
from jax.experimental import pallas as pl
from jax.experimental.pallas import tpu as pltpu
from jax.experimental.pallas import tpu_sc as plsc

RES = 100000
N = 8388608
V0 = 0.0
VR = 1.0
START_VALUE = 0.5
UPDATE_WEIGHT = 1.0 / 37500000.0
CW = float((1.0 - UPDATE_WEIGHT) ** float(N))

NC = 2
NS = 16
NW = NC * NS
NPW = N // NW
CH = 8192
NCHUNK = NPW // CH
CROWS = CH // 128
HP = 100096
ZS = HP // NS

P2R = HP // 128
P2C = 128


def _sc_hist_body(stat_hbm, dyn_hbm, score_hbm, out_hbm, hist_sh,
                  stat_v, dyn_v, score_v, vals_v, idx_v, zbuf,
                  in_sems, sc_sems):
    c = lax.axis_index("c")
    s = lax.axis_index("s")
    wid = s * NC + c

    @pl.loop(0, ZS // 16)
    def _zero(i):
        zbuf[pl.ds(i * 16, 16)] = jnp.zeros((16,), jnp.float32)

    pltpu.sync_copy(zbuf, hist_sh.at[pl.ds(s * ZS, ZS)])
    plsc.subcore_barrier()

    base = wid * NPW
    inputs = (stat_hbm, dyn_hbm, score_hbm)

    def start_inputs(b, k):
        off = base + k * CH
        for src, dst in zip(inputs, (stat_v[b], dyn_v[b], score_v[b])):
            pltpu.async_copy(src.at[pl.ds(off, CH)], dst, in_sems[b])

    def wait_inputs(b, k):
        off = base + k * CH
        for src, dst in zip(inputs, (stat_v[b], dyn_v[b], score_v[b])):
            pltpu.make_async_copy(src.at[pl.ds(off, CH)], dst, in_sems[b]).wait()

    def wait_scatter(b):
        pltpu.make_async_copy(
            vals_v[b], hist_sh.at[idx_v[b]], sc_sems[b]).wait()

    start_inputs(0, 0)
    start_inputs(1, 1)

    @pl.loop(0, NCHUNK // 2)
    def _round(o):
        for b in range(2):
            k = o * 2 + b
            wait_inputs(b, k)

            @pl.when(k >= 2)
            def _():
                wait_scatter(b)

            @pl.loop(0, CH // 16, unroll=8)
            def _vec(i):
                st = stat_v[b][pl.ds(i * 16, 16)]
                dy = dyn_v[b][pl.ds(i * 16, 16)]
                sc = score_v[b][pl.ds(i * 16, 16)]
                vals = st - dy
                idxi = (sc * float(RES)).astype(jnp.int32)
                idxi = jnp.minimum(idxi, RES - 1)
                vals_v[b][pl.ds(i * 16, 16)] = vals
                idx_v[b][pl.ds(i * 16, 16)] = idxi

            pltpu.async_copy(vals_v[b], hist_sh.at[idx_v[b]], sc_sems[b],
                             add=True)

            @pl.when(k + 2 < NCHUNK)
            def _():
                start_inputs(b, k + 2)

    wait_scatter(0)
    wait_scatter(1)
    plsc.subcore_barrier()

    pltpu.sync_copy(hist_sh.at[pl.ds(s * ZS, ZS)], zbuf)
    pltpu.sync_copy(zbuf, out_hbm.at[pl.ds(c * HP + s * ZS, ZS)])


@functools.cache
def _sc_hist():
    return pl.kernel(
        _sc_hist_body,
        out_type=jax.ShapeDtypeStruct((NC * HP,), jnp.float32),
        mesh=plsc.VectorSubcoreMesh(core_axis_name="c", subcore_axis_name="s",
                                    num_cores=NC, num_subcores=NS),
        scratch_types=[
            pltpu.VMEM_SHARED((HP,), jnp.float32),
            [pltpu.VMEM((CH,), jnp.float32) for _ in range(2)],
            [pltpu.VMEM((CH,), jnp.float32) for _ in range(2)],
            [pltpu.VMEM((CH,), jnp.float32) for _ in range(2)],
            [pltpu.VMEM((CH,), jnp.float32) for _ in range(2)],
            [pltpu.VMEM((CH,), jnp.int32) for _ in range(2)],
            pltpu.VMEM((ZS,), jnp.float32),
            [pltpu.SemaphoreType.DMA for _ in range(2)],
            [pltpu.SemaphoreType.DMA for _ in range(2)],
        ],
    )


def _shift_right_cols(x, s):
    return jnp.concatenate(
        [jnp.zeros((x.shape[0], s), jnp.float32), x[:, : x.shape[1] - s]], axis=1)


def _shift_down_rows(x, s):
    return jnp.concatenate(
        [jnp.zeros((s, x.shape[1]), jnp.float32), x[: x.shape[0] - s, :]], axis=0)


def _tc_thresh_body(parts_ref, mov_ref, bias_ref, out_ref):
    h = parts_ref[0] + parts_ref[1]
    imp = mov_ref[...] * CW + (1.0 - CW) * h

    x = imp
    for sft in (1, 2, 4, 8, 16, 32, 64):
        x = x + _shift_right_cols(x, sft)
    rowtot = x[:, P2C - 1:P2C]

    t = _shift_down_rows(rowtot, 1)
    for sft in (1, 2, 4, 8, 16, 32, 64, 128, 256, 512):
        t = t + _shift_down_rows(t, sft)

    prefix = x + t

    flat = (lax.broadcasted_iota(jnp.int32, (P2R, P2C), 0) * P2C
            + lax.broadcasted_iota(jnp.int32, (P2R, P2C), 1))
    valid = flat < RES
    pv = jnp.where(valid, prefix, jnp.inf)
    best = jnp.minimum(jnp.min(pv), 0.0)

    maskv = valid & (prefix == best)
    cnt = (jnp.sum(maskv.astype(jnp.float32))
           + jnp.where(best == 0.0, 1.0, 0.0))
    sidx = jnp.sum(jnp.where(maskv, (flat + 1).astype(jnp.float32), 0.0))
    avg = sidx / cnt
    thr = avg * VR / float(RES) + V0

    new_bias = bias_ref[0, 0] * CW + (1.0 - CW)
    out_ref[0, 0] = jnp.where(new_bias > 0.0, thr, jnp.float32(START_VALUE))


def kernel(epes_stat_flow, epes_dyn_flow, dynamicness_scores,
           moving_average_importance, bias_counter):
    parts = _sc_hist()(epes_stat_flow, epes_dyn_flow, dynamicness_scores)

    parts_p = parts.reshape(NC, P2R, P2C)
    mov_p = jnp.pad(moving_average_importance, (0, HP - RES))
    mov_p = mov_p.reshape(P2R, P2C)
    bias = jnp.reshape(bias_counter, (1, 1))

    out = pl.pallas_call(
        _tc_thresh_body,
        out_shape=jax.ShapeDtypeStruct((1, 1), jnp.float32),
        in_specs=[
            pl.BlockSpec(memory_space=pltpu.VMEM),
            pl.BlockSpec(memory_space=pltpu.VMEM),
            pl.BlockSpec(memory_space=pltpu.SMEM),
        ],
        out_specs=pl.BlockSpec(memory_space=pltpu.SMEM),
    )(parts_p, mov_p, bias)
    return out[0, 0]
